# baseline (speedup 1.0000x reference)
import functools

import jax
import jax.numpy as jnp
from jax.experimental import pallas as pl
from jax.experimental.pallas import tpu as pltpu


def _stage1_body(x_ref, w1t_ref, b1_ref, w2t_ref, b2_ref,
                 pmax_ref, amax_ref, col_ref):
    h = jnp.maximum(
        jnp.dot(x_ref[...], w1t_ref[...], preferred_element_type=jnp.float32)
        + b1_ref[...], 0.0)
    logits = (jnp.dot(h, w2t_ref[...], preferred_element_type=jnp.float32)
              + b2_ref[...])
    br, ne = logits.shape
    m = jnp.max(logits, axis=1, keepdims=True)
    e = jnp.exp(logits - m)
    s = jnp.sum(e, axis=1, keepdims=True)
    pmax = 1.0 / s
    amax = jnp.argmax(logits, axis=1).astype(jnp.int32)[:, None]
    onehot = jax.lax.broadcasted_iota(jnp.int32, (br, ne), 1) == amax
    masked = jnp.where(onehot, pmax, 0.0)
    pmax_ref[...] = pmax
    amax_ref[...] = amax
    col_ref[...] = jnp.sum(masked, axis=0)[None, None, :]


def _stage2_body(pmax_ref, amax_ref, col_ref, out_ref, *, capacity):
    cols = col_ref[...]
    denom = jnp.sum(cols, axis=(0, 1))[None, :] + 0.0001
    t = pmax_ref[...] * (capacity / denom)
    br, ne = t.shape
    onehot = (jax.lax.broadcasted_iota(jnp.int32, (br, ne), 1)
              == amax_ref[...])
    out_ref[...] = jnp.where(onehot, t, 0.0)


def kernel(x, W1, b1, W2, b2):
    n, d = x.shape
    nh = W1.shape[0]
    ne = W2.shape[0]
    br = min(4096, n)
    nb = n // br
    capacity = float(n)

    w1t = W1.T
    w2t = W2.T
    b1r = b1.reshape(1, nh)
    b2r = b2.reshape(1, ne)

    pmax, amax, colpart = pl.pallas_call(
        _stage1_body,
        grid=(nb,),
        in_specs=[
            pl.BlockSpec((br, d), lambda i: (i, 0)),
            pl.BlockSpec((d, nh), lambda i: (0, 0)),
            pl.BlockSpec((1, nh), lambda i: (0, 0)),
            pl.BlockSpec((nh, ne), lambda i: (0, 0)),
            pl.BlockSpec((1, ne), lambda i: (0, 0)),
        ],
        out_specs=[
            pl.BlockSpec((br, 1), lambda i: (i, 0)),
            pl.BlockSpec((br, 1), lambda i: (i, 0)),
            pl.BlockSpec((1, 1, ne), lambda i: (i, 0, 0)),
        ],
        out_shape=[
            jax.ShapeDtypeStruct((n, 1), jnp.float32),
            jax.ShapeDtypeStruct((n, 1), jnp.int32),
            jax.ShapeDtypeStruct((nb, 1, ne), jnp.float32),
        ],
        compiler_params=pltpu.CompilerParams(
            dimension_semantics=("parallel",)),
    )(x, w1t, b1r, w2t, b2r)

    br2 = min(4096, n)
    nb2 = n // br2
    out = pl.pallas_call(
        functools.partial(_stage2_body, capacity=capacity),
        grid=(nb2,),
        in_specs=[
            pl.BlockSpec((br2, 1), lambda i: (i, 0)),
            pl.BlockSpec((br2, 1), lambda i: (i, 0)),
            pl.BlockSpec((nb, 1, ne), lambda i: (0, 0, 0)),
        ],
        out_specs=pl.BlockSpec((br2, ne), lambda i: (i, 0)),
        out_shape=jax.ShapeDtypeStruct((n, ne), jnp.float32),
        compiler_params=pltpu.CompilerParams(
            dimension_semantics=("parallel",)),
    )(pmax, amax, colpart)

    return out


# speedup vs baseline: 1.2190x; 1.2190x over previous
import jax
import jax.numpy as jnp
from jax.experimental import pallas as pl
from jax.experimental.pallas import tpu as pltpu


def _fused_body(x_ref, w1t_ref, b1_ref, w2t_ref, b2_ref, out_ref,
                pmax_s, amax_s, col_s):
    k = pl.program_id(0)
    i = pl.program_id(1)
    br, nb = pmax_s.shape
    ne = out_ref.shape[1]
    capacity = jnp.float32(br * nb)
    col_iota = jax.lax.broadcasted_iota(jnp.int32, (br, nb), 1)

    @pl.when(k == 0)
    def _phase_a():
        h = jnp.maximum(
            jnp.dot(x_ref[...], w1t_ref[...],
                    preferred_element_type=jnp.float32) + b1_ref[...], 0.0)
        logits = (jnp.dot(h, w2t_ref[...],
                          preferred_element_type=jnp.float32) + b2_ref[...])
        m = jnp.max(logits, axis=1, keepdims=True)
        e = jnp.exp(logits - m)
        s = jnp.sum(e, axis=1, keepdims=True)
        pmax = 1.0 / s
        amax = jnp.argmax(logits, axis=1).astype(jnp.int32)[:, None]
        onehot = jax.lax.broadcasted_iota(jnp.int32, (br, ne), 1) == amax
        masked = jnp.where(onehot, pmax, 0.0)
        pmax_s[...] = jnp.where(col_iota == i, pmax, pmax_s[...])
        amax_s[...] = jnp.where(col_iota == i, amax, amax_s[...])
        colpart = jnp.sum(masked, axis=0)[None, :]

        @pl.when(i == 0)
        def _():
            col_s[...] = colpart

        @pl.when(i > 0)
        def _():
            col_s[...] = col_s[...] + colpart

    @pl.when(k == 1)
    def _phase_b():
        denom = col_s[...] + 0.0001
        sel = col_iota == i
        pmax = jnp.sum(jnp.where(sel, pmax_s[...], 0.0), axis=1,
                       keepdims=True)
        amax = jnp.sum(jnp.where(sel, amax_s[...], 0), axis=1,
                       keepdims=True)
        t = pmax * (capacity / denom)
        onehot = (jax.lax.broadcasted_iota(jnp.int32, (br, ne), 1) == amax)
        out_ref[...] = jnp.where(onehot, t, 0.0)


def kernel(x, W1, b1, W2, b2):
    n, d = x.shape
    nh = W1.shape[0]
    ne = W2.shape[0]
    br = min(4096, n)
    nb = n // br

    w1t = W1.T
    w2t = W2.T
    b1r = b1.reshape(1, nh)
    b2r = b2.reshape(1, ne)

    out = pl.pallas_call(
        _fused_body,
        grid=(2, nb),
        in_specs=[
            pl.BlockSpec((br, d), lambda k, i: (jnp.where(k == 0, i, nb - 1), 0)),
            pl.BlockSpec((d, nh), lambda k, i: (0, 0)),
            pl.BlockSpec((1, nh), lambda k, i: (0, 0)),
            pl.BlockSpec((nh, ne), lambda k, i: (0, 0)),
            pl.BlockSpec((1, ne), lambda k, i: (0, 0)),
        ],
        out_specs=pl.BlockSpec((br, ne),
                               lambda k, i: (jnp.where(k == 0, 0, i), 0)),
        out_shape=jax.ShapeDtypeStruct((n, ne), jnp.float32),
        scratch_shapes=[
            pltpu.VMEM((br, nb), jnp.float32),
            pltpu.VMEM((br, nb), jnp.int32),
            pltpu.VMEM((1, ne), jnp.float32),
        ],
        compiler_params=pltpu.CompilerParams(
            dimension_semantics=("arbitrary", "arbitrary")),
    )(x, w1t, b1r, w2t, b2r)

    return out
